# baseline (speedup 1.0000x reference)
import functools

import jax
import jax.numpy as jnp
from jax import lax
from jax.experimental import pallas as pl
from jax.experimental.pallas import tpu as pltpu
from jax.experimental.pallas import tpu_sc as plsc

N = 100000
NP = 100352
RPT = NP // 16
E = 6400000
NC, NS = 2, 16
NT = NC * NS
CH = 5
EB = CH * 128
RT = 1570
EP = NT * RT * 128
XR = CH
NB2 = RT // (2 * CH)

_MESH = plsc.VectorSubcoreMesh(core_axis_name="c", subcore_axis_name="s",
                               num_cores=NC, num_subcores=NS)
_SC_PARAMS = pltpu.CompilerParams(use_tc_tiling_on_sc=False)


def _make_edge_pass(w):
  def body(u_hbm, src_hbm, dst_hbm, z_hbm, out_hbm, src_v0, dst_v0, src_v1,
           dst_v1, rows0, rows1, acc, gA, gB, sA, sB):
    c = lax.axis_index("c")
    s = lax.axis_index("s")
    wid = c * NS + s
    pltpu.sync_copy(z_hbm, acc.at[pl.ds(s * RPT, RPT)])
    plsc.subcore_barrier()

    def load_idx(sv, dv, row0):
      pltpu.sync_copy(src_hbm.at[pl.ds(row0, CH)], sv)
      pltpu.sync_copy(dst_hbm.at[pl.ds(row0, CH)], dv)

    def fire_gathers(sv, rows, sem):
      for j in range(CH):
        pltpu.async_copy(u_hbm.at[sv.at[j]],
                         rows.at[pl.ds(j * 128, 128)], sem)

    def fire_scatters(rows, dv, sem):
      return [
          pltpu.async_copy(rows.at[pl.ds(j * 128, 128)],
                           acc.at[dv.at[j]], sem, add=True)
          for j in range(CH)
      ]

    def drain(sem, rows):
      pltpu.make_async_copy(u_hbm.at[pl.ds(0, EB)], rows, sem).wait()

    load_idx(src_v1, dst_v1, NT * RT)
    for d in fire_scatters(rows1, dst_v1, sB):
      pass
    load_idx(src_v0, dst_v0, wid * RT)
    fire_gathers(src_v0, rows0, gA)

    def step(m, carry):
      rowB = wid * RT + (2 * m + 1) * CH
      rowA2 = wid * RT + (2 * m + 2) * CH
      drain(gA, rows0)
      sdA = fire_scatters(rows0, dst_v0, sA)
      drain(sB, rows1)
      load_idx(src_v1, dst_v1, rowB)
      fire_gathers(src_v1, rows1, gB)
      for d in sdA:
        d.wait()
      load_idx(src_v0, dst_v0, rowA2)
      fire_gathers(src_v0, rows0, gA)
      drain(gB, rows1)
      fire_scatters(rows1, dst_v1, sB)
      return carry

    lax.fori_loop(0, NB2, step, 0)
    drain(gA, rows0)
    drain(sB, rows1)
    plsc.subcore_barrier()
    pltpu.sync_copy(acc.at[pl.ds(s * RPT, RPT)],
                    out_hbm.at[pl.ds(c * NP + s * RPT, RPT)])

  return pl.kernel(
      body,
      out_type=jax.ShapeDtypeStruct((NC * NP, w), jnp.float32),
      mesh=_MESH,
      scratch_types=[
          pltpu.VMEM((CH, 128), jnp.int32),
          pltpu.VMEM((CH, 128), jnp.int32),
          pltpu.VMEM((CH, 128), jnp.int32),
          pltpu.VMEM((CH, 128), jnp.int32),
          pltpu.VMEM((EB, w), jnp.float32),
          pltpu.VMEM((EB, w), jnp.float32),
          pltpu.VMEM_SHARED((NP, w), jnp.float32),
          pltpu.SemaphoreType.DMA,
          pltpu.SemaphoreType.DMA,
          pltpu.SemaphoreType.DMA,
          pltpu.SemaphoreType.DMA,
      ],
      compiler_params=_SC_PARAMS)


def _make_deg_pass():
  def body(dst_hbm, z_hbm, out_hbm, dst_v, ones_v, acc):
    c = lax.axis_index("c")
    s = lax.axis_index("s")
    wid = c * NS + s
    for i in range(8):
      ones_v[pl.ds(16 * i, 16)] = jnp.ones((16,), jnp.float32)
    pltpu.sync_copy(z_hbm, acc.at[pl.ds(s * RPT, RPT)])
    plsc.subcore_barrier()

    def step(m, carry):
      row0 = wid * RT + m * CH
      pltpu.sync_copy(dst_hbm.at[pl.ds(row0, CH)], dst_v)
      for j in range(CH):
        pltpu.sync_copy(ones_v, acc.at[dst_v.at[j]], add=True)
      return carry

    lax.fori_loop(0, RT // CH, step, 0)
    plsc.subcore_barrier()
    pltpu.sync_copy(acc.at[pl.ds(s * RPT, RPT)],
                    out_hbm.at[pl.ds(c * NP + s * RPT, RPT)])

  return pl.kernel(
      body,
      out_type=jax.ShapeDtypeStruct((NC * NP,), jnp.float32),
      mesh=_MESH,
      scratch_types=[
          pltpu.VMEM((CH, 128), jnp.int32),
          pltpu.VMEM((128,), jnp.float32),
          pltpu.VMEM_SHARED((NP,), jnp.float32),
      ],
      compiler_params=_SC_PARAMS)


_edge_pass8 = _make_edge_pass(8)
_edge_pass16 = _make_edge_pass(16)
_deg_pass = _make_deg_pass()

BLK = 1024
GRID = NP // BLK


def _row_spec(w):
  return pl.BlockSpec((BLK, w), lambda i: (i, 0))


def _full_spec(shape):
  return pl.BlockSpec(shape, lambda i: (0, 0))


def _stage_a(d0, d1, xp):
  def body(d0_ref, d1_ref, x_ref, dinv_ref, u1_ref):
    deg = d0_ref[...] + d1_ref[...] + 1.0
    dinv = lax.rsqrt(deg)
    dinv_ref[...] = dinv
    u1_ref[...] = dinv * x_ref[...]

  return pl.pallas_call(
      body,
      grid=(GRID,),
      in_specs=[_row_spec(1), _row_spec(1), _row_spec(8)],
      out_specs=[_row_spec(1), _row_spec(8)],
      out_shape=[jax.ShapeDtypeStruct((NP, 1), jnp.float32),
                 jax.ShapeDtypeStruct((NP, 8), jnp.float32)],
  )(d0, d1, xp)


def _stage_b(s0, s1, u1, dinv, W1p, b1):
  def body(s0_ref, s1_ref, u_ref, dinv_ref, w_ref, b_ref, u2_ref):
    dinv = dinv_ref[...]
    a = dinv * (s0_ref[...] + s1_ref[...] + u_ref[...])
    h = jnp.dot(a, w_ref[...], preferred_element_type=jnp.float32)
    h = jnp.maximum(h + b_ref[...], 0.0)
    u2_ref[...] = dinv * h

  return pl.pallas_call(
      body,
      grid=(GRID,),
      in_specs=[_row_spec(8), _row_spec(8), _row_spec(8), _row_spec(1),
                _full_spec((8, 16)), _full_spec((1, 16))],
      out_specs=_row_spec(16),
      out_shape=jax.ShapeDtypeStruct((NP, 16), jnp.float32),
  )(s0, s1, u1, dinv, W1p, b1)


def _stage_c(s0, s1, u2, dinv, W2, b2, W3p):
  def body(s0_ref, s1_ref, u_ref, dinv_ref, w2_ref, b2_ref, w3_ref, u3_ref):
    dinv = dinv_ref[...]
    a = dinv * (s0_ref[...] + s1_ref[...] + u_ref[...])
    h = jnp.dot(a, w2_ref[...], preferred_element_type=jnp.float32)
    h = jnp.maximum(h + b2_ref[...], 0.0)
    g = jnp.dot(h, w3_ref[...], preferred_element_type=jnp.float32)
    u3_ref[...] = dinv * g

  return pl.pallas_call(
      body,
      grid=(GRID,),
      in_specs=[_row_spec(16), _row_spec(16), _row_spec(16), _row_spec(1),
                _full_spec((16, 32)), _full_spec((1, 32)),
                _full_spec((32, 8))],
      out_specs=_row_spec(8),
      out_shape=jax.ShapeDtypeStruct((NP, 8), jnp.float32),
  )(s0, s1, u2, dinv, W2, b2, W3p)


def _stage_d(s0, s1, u3, dinv, b3p):
  def body(s0_ref, s1_ref, u_ref, dinv_ref, b_ref, o_ref):
    o_ref[...] = (dinv_ref[...] * (s0_ref[...] + s1_ref[...] + u_ref[...])
                  + b_ref[...])

  return pl.pallas_call(
      body,
      grid=(GRID,),
      in_specs=[_row_spec(8), _row_spec(8), _row_spec(8), _row_spec(1),
                _full_spec((1, 8))],
      out_specs=_row_spec(8),
      out_shape=jax.ShapeDtypeStruct((NP, 8), jnp.float32),
  )(s0, s1, u3, dinv, b3p)


def kernel(x, edge_index, W1, b1, W2, b2, W3, b3):
  src = edge_index[0].astype(jnp.int32)
  dst = edge_index[1].astype(jnp.int32)
  padv = jnp.full((EP + XR * 128 - E,), N, jnp.int32)
  src2 = jnp.concatenate([src, padv]).reshape(EP // 128 + XR, 128)
  dst2 = jnp.concatenate([dst, padv]).reshape(EP // 128 + XR, 128)
  xp = jnp.zeros((NP, 8), jnp.float32).at[:N, :3].set(x)
  W1p = jnp.zeros((8, 16), jnp.float32).at[:3].set(W1)
  W3p = jnp.zeros((32, 8), jnp.float32).at[:, :2].set(W3)
  b3p = jnp.zeros((1, 8), jnp.float32).at[0, :2].set(b3)
  z1 = jnp.zeros((RPT,), jnp.float32)
  z8 = jnp.zeros((RPT, 8), jnp.float32)
  z16 = jnp.zeros((RPT, 16), jnp.float32)

  degs = _deg_pass(dst2, z1)
  d0 = degs[:NP].reshape(NP, 1)
  d1 = degs[NP:].reshape(NP, 1)
  dinv, u1 = _stage_a(d0, d1, xp)
  s1 = _edge_pass8(u1, src2, dst2, z8)
  u2 = _stage_b(s1[:NP], s1[NP:], u1, dinv, W1p, b1.reshape(1, 16))
  s2 = _edge_pass16(u2, src2, dst2, z16)
  u3 = _stage_c(s2[:NP], s2[NP:], u2, dinv, W2, b2.reshape(1, 32), W3p)
  s3 = _edge_pass8(u3, src2, dst2, z8)
  outp = _stage_d(s3[:NP], s3[NP:], u3, dinv, b3p)
  return outp[:N, :2]


# speedup vs baseline: 1.1376x; 1.1376x over previous
import functools

import jax
import jax.numpy as jnp
from jax import lax
from jax.experimental import pallas as pl
from jax.experimental.pallas import tpu as pltpu
from jax.experimental.pallas import tpu_sc as plsc

N = 100000
NP = 100352
RPT = NP // 16
E = 6400000
NC, NS = 2, 16
NT = NC * NS
DCH = 5
RTD = 1570
R_ALL = NT * 1572 + 6
EP = R_ALL * 128
NBD = RTD // (2 * DCH)

_MESH = plsc.VectorSubcoreMesh(core_axis_name="c", subcore_axis_name="s",
                               num_cores=NC, num_subcores=NS)
_SC_PARAMS = pltpu.CompilerParams(use_tc_tiling_on_sc=False)


def _make_edge_pass(w, CH, RT):
  EB = CH * 128
  NB2 = RT // (2 * CH)

  def body(u_hbm, src_hbm, dst_hbm, z_hbm, out_hbm, src_v0, dst_v0, src_v1,
           dst_v1, rows0, rows1, acc, gA, gB, sA, sB):
    c = lax.axis_index("c")
    s = lax.axis_index("s")
    wid = c * NS + s
    pltpu.sync_copy(z_hbm, acc.at[pl.ds(s * RPT, RPT)])
    plsc.subcore_barrier()

    def load_idx(sv, dv, row0):
      pltpu.sync_copy(src_hbm.at[pl.ds(row0, CH)], sv)
      pltpu.sync_copy(dst_hbm.at[pl.ds(row0, CH)], dv)

    def fire_gathers(sv, rows, sem):
      for j in range(CH):
        pltpu.async_copy(u_hbm.at[sv.at[j]],
                         rows.at[pl.ds(j * 128, 128)], sem)

    def fire_scatters(rows, dv, sem):
      return [
          pltpu.async_copy(rows.at[pl.ds(j * 128, 128)],
                           acc.at[dv.at[j]], sem, add=True)
          for j in range(CH)
      ]

    def drain(sem, rows):
      pltpu.make_async_copy(u_hbm.at[pl.ds(0, EB)], rows, sem).wait()

    load_idx(src_v1, dst_v1, NT * RT)
    for d in fire_scatters(rows1, dst_v1, sB):
      pass
    load_idx(src_v0, dst_v0, wid * RT)
    fire_gathers(src_v0, rows0, gA)

    def step(m, carry):
      rowB = wid * RT + (2 * m + 1) * CH
      rowA2 = wid * RT + (2 * m + 2) * CH
      drain(gA, rows0)
      sdA = fire_scatters(rows0, dst_v0, sA)
      drain(sB, rows1)
      load_idx(src_v1, dst_v1, rowB)
      fire_gathers(src_v1, rows1, gB)
      for d in sdA:
        d.wait()
      load_idx(src_v0, dst_v0, rowA2)
      fire_gathers(src_v0, rows0, gA)
      drain(gB, rows1)
      fire_scatters(rows1, dst_v1, sB)
      return carry

    lax.fori_loop(0, NB2, step, 0)
    drain(gA, rows0)
    drain(sB, rows1)
    plsc.subcore_barrier()
    pltpu.sync_copy(acc.at[pl.ds(s * RPT, RPT)],
                    out_hbm.at[pl.ds(c * NP + s * RPT, RPT)])

  return pl.kernel(
      body,
      out_type=jax.ShapeDtypeStruct((NC * NP, w), jnp.float32),
      mesh=_MESH,
      scratch_types=[
          pltpu.VMEM((CH, 128), jnp.int32),
          pltpu.VMEM((CH, 128), jnp.int32),
          pltpu.VMEM((CH, 128), jnp.int32),
          pltpu.VMEM((CH, 128), jnp.int32),
          pltpu.VMEM((EB, w), jnp.float32),
          pltpu.VMEM((EB, w), jnp.float32),
          pltpu.VMEM_SHARED((NP, w), jnp.float32),
          pltpu.SemaphoreType.DMA,
          pltpu.SemaphoreType.DMA,
          pltpu.SemaphoreType.DMA,
          pltpu.SemaphoreType.DMA,
      ],
      compiler_params=_SC_PARAMS)


def _make_deg_pass():
  def body(dst_hbm, z_hbm, out_hbm, dst_v0, dst_v1, ones_v, acc, sA, sB):
    c = lax.axis_index("c")
    s = lax.axis_index("s")
    wid = c * NS + s
    for i in range(8):
      ones_v[pl.ds(16 * i, 16)] = jnp.ones((16,), jnp.float32)
    pltpu.sync_copy(z_hbm, acc.at[pl.ds(s * RPT, RPT)])
    plsc.subcore_barrier()

    def fire(dv, sem):
      for j in range(DCH):
        pltpu.async_copy(ones_v, acc.at[dv.at[j]], sem, add=True)

    def drain(sem):
      for _ in range(DCH):
        pltpu.make_async_copy(z_hbm.at[pl.ds(0, 128)], ones_v, sem).wait()

    pltpu.sync_copy(dst_hbm.at[pl.ds(NT * RTD, DCH)], dst_v1)
    fire(dst_v1, sB)
    pltpu.sync_copy(dst_hbm.at[pl.ds(wid * RTD, DCH)], dst_v0)

    def step(m, carry):
      base = wid * RTD + 2 * m * DCH
      fire(dst_v0, sA)
      drain(sB)
      pltpu.sync_copy(dst_hbm.at[pl.ds(base + DCH, DCH)], dst_v1)
      fire(dst_v1, sB)
      drain(sA)
      pltpu.sync_copy(dst_hbm.at[pl.ds(base + 2 * DCH, DCH)], dst_v0)
      return carry

    lax.fori_loop(0, NBD, step, 0)
    drain(sB)
    plsc.subcore_barrier()
    pltpu.sync_copy(acc.at[pl.ds(s * RPT, RPT)],
                    out_hbm.at[pl.ds(c * NP + s * RPT, RPT)])

  return pl.kernel(
      body,
      out_type=jax.ShapeDtypeStruct((NC * NP,), jnp.float32),
      mesh=_MESH,
      scratch_types=[
          pltpu.VMEM((DCH, 128), jnp.int32),
          pltpu.VMEM((DCH, 128), jnp.int32),
          pltpu.VMEM((128,), jnp.float32),
          pltpu.VMEM_SHARED((NP,), jnp.float32),
          pltpu.SemaphoreType.DMA,
          pltpu.SemaphoreType.DMA,
      ],
      compiler_params=_SC_PARAMS)


_edge_pass8 = _make_edge_pass(8, 8, 1568)
_edge_pass16 = _make_edge_pass(16, 6, 1572)
_deg_pass = _make_deg_pass()

BLK = 1024
GRID = NP // BLK


def _row_spec(w):
  return pl.BlockSpec((BLK, w), lambda i: (i, 0))


def _full_spec(shape):
  return pl.BlockSpec(shape, lambda i: (0, 0))


def _stage_a(d0, d1, xp):
  def body(d0_ref, d1_ref, x_ref, dinv_ref, u1_ref):
    deg = d0_ref[...] + d1_ref[...] + 1.0
    dinv = lax.rsqrt(deg)
    dinv_ref[...] = dinv
    u1_ref[...] = dinv * x_ref[...]

  return pl.pallas_call(
      body,
      grid=(GRID,),
      in_specs=[_row_spec(1), _row_spec(1), _row_spec(8)],
      out_specs=[_row_spec(1), _row_spec(8)],
      out_shape=[jax.ShapeDtypeStruct((NP, 1), jnp.float32),
                 jax.ShapeDtypeStruct((NP, 8), jnp.float32)],
  )(d0, d1, xp)


def _stage_b(s0, s1, u1, dinv, W1p, b1):
  def body(s0_ref, s1_ref, u_ref, dinv_ref, w_ref, b_ref, u2_ref):
    dinv = dinv_ref[...]
    a = dinv * (s0_ref[...] + s1_ref[...] + u_ref[...])
    h = jnp.dot(a, w_ref[...], preferred_element_type=jnp.float32)
    h = jnp.maximum(h + b_ref[...], 0.0)
    u2_ref[...] = dinv * h

  return pl.pallas_call(
      body,
      grid=(GRID,),
      in_specs=[_row_spec(8), _row_spec(8), _row_spec(8), _row_spec(1),
                _full_spec((8, 16)), _full_spec((1, 16))],
      out_specs=_row_spec(16),
      out_shape=jax.ShapeDtypeStruct((NP, 16), jnp.float32),
  )(s0, s1, u1, dinv, W1p, b1)


def _stage_c(s0, s1, u2, dinv, W2, b2, W3p):
  def body(s0_ref, s1_ref, u_ref, dinv_ref, w2_ref, b2_ref, w3_ref, u3_ref):
    dinv = dinv_ref[...]
    a = dinv * (s0_ref[...] + s1_ref[...] + u_ref[...])
    h = jnp.dot(a, w2_ref[...], preferred_element_type=jnp.float32)
    h = jnp.maximum(h + b2_ref[...], 0.0)
    g = jnp.dot(h, w3_ref[...], preferred_element_type=jnp.float32)
    u3_ref[...] = dinv * g

  return pl.pallas_call(
      body,
      grid=(GRID,),
      in_specs=[_row_spec(16), _row_spec(16), _row_spec(16), _row_spec(1),
                _full_spec((16, 32)), _full_spec((1, 32)),
                _full_spec((32, 8))],
      out_specs=_row_spec(8),
      out_shape=jax.ShapeDtypeStruct((NP, 8), jnp.float32),
  )(s0, s1, u2, dinv, W2, b2, W3p)


def _stage_d(s0, s1, u3, dinv, b3p):
  def body(s0_ref, s1_ref, u_ref, dinv_ref, b_ref, o_ref):
    o_ref[...] = (dinv_ref[...] * (s0_ref[...] + s1_ref[...] + u_ref[...])
                  + b_ref[...])

  return pl.pallas_call(
      body,
      grid=(GRID,),
      in_specs=[_row_spec(8), _row_spec(8), _row_spec(8), _row_spec(1),
                _full_spec((1, 8))],
      out_specs=_row_spec(8),
      out_shape=jax.ShapeDtypeStruct((NP, 8), jnp.float32),
  )(s0, s1, u3, dinv, b3p)


def kernel(x, edge_index, W1, b1, W2, b2, W3, b3):
  src = edge_index[0].astype(jnp.int32)
  dst = edge_index[1].astype(jnp.int32)
  padv = jnp.full((EP - E,), N, jnp.int32)
  src2 = jnp.concatenate([src, padv]).reshape(R_ALL, 128)
  dst2 = jnp.concatenate([dst, padv]).reshape(R_ALL, 128)
  xp = jnp.zeros((NP, 8), jnp.float32).at[:N, :3].set(x)
  W1p = jnp.zeros((8, 16), jnp.float32).at[:3].set(W1)
  W3p = jnp.zeros((32, 8), jnp.float32).at[:, :2].set(W3)
  b3p = jnp.zeros((1, 8), jnp.float32).at[0, :2].set(b3)
  z1 = jnp.zeros((RPT,), jnp.float32)
  z8 = jnp.zeros((RPT, 8), jnp.float32)
  z16 = jnp.zeros((RPT, 16), jnp.float32)

  degs = _deg_pass(dst2, z1)
  d0 = degs[:NP].reshape(NP, 1)
  d1 = degs[NP:].reshape(NP, 1)
  dinv, u1 = _stage_a(d0, d1, xp)
  s1 = _edge_pass8(u1, src2, dst2, z8)
  u2 = _stage_b(s1[:NP], s1[NP:], u1, dinv, W1p, b1.reshape(1, 16))
  s2 = _edge_pass16(u2, src2, dst2, z16)
  u3 = _stage_c(s2[:NP], s2[NP:], u2, dinv, W2, b2.reshape(1, 32), W3p)
  s3 = _edge_pass8(u3, src2, dst2, z8)
  outp = _stage_d(s3[:NP], s3[NP:], u3, dinv, b3p)
  return outp[:N, :2]
